# Initial kernel scaffold; baseline (speedup 1.0000x reference)
import jax
import jax.numpy as jnp
from jax.experimental import pallas as pl


def kernel(x, edge_index, W, b, We, be):
    raise NotImplementedError("write your pallas kernel here")


# speedup vs baseline: 31.0393x; 31.0393x over previous
import functools

import jax
import jax.numpy as jnp
from jax import lax
from jax.experimental import pallas as pl
from jax.experimental.pallas import tpu as pltpu
from jax.experimental.pallas import tpu_sc as plsc

N = 10000
E = 320000
D = 128
F = D // 2
NP = 10240
NC = 2
NS = 16
ROW = 125
EROWS = E // ROW

_MESH = dict(core_axis_name="c", subcore_axis_name="s")


def _deg_body(dst2d_hbm, out_hbm, idx_v, ones_v, zrow_v, hist_sh):
    c = lax.axis_index("c")
    s = lax.axis_index("s")

    zero16 = jnp.zeros((16,), jnp.float32)
    one16 = jnp.ones((16,), jnp.float32)
    for i in range(40):
        zrow_v[pl.ds(i * 16, 16)] = zero16
    for i in range(8):
        ones_v[pl.ds(i * 16, 16)] = one16

    pltpu.sync_copy(zrow_v, hist_sh.at[pl.ds(s * 640, 640)])
    plsc.subcore_barrier()

    rbase = (c * NS + s) * 80
    pltpu.sync_copy(dst2d_hbm.at[pl.ds(rbase, 80)], idx_v)

    def step(j, _):
        pltpu.sync_copy(ones_v.at[pl.ds(0, ROW)], hist_sh.at[idx_v.at[j]],
                        add=True)
        return 0

    lax.fori_loop(0, 80, step, 0)
    plsc.subcore_barrier()
    pltpu.sync_copy(hist_sh.at[pl.ds(s * 640, 640)],
                    out_hbm.at[c, pl.ds(s * 640, 640)])


_deg_call = functools.partial(
    pl.kernel,
    out_type=jax.ShapeDtypeStruct((NC, NP), jnp.float32),
    mesh=plsc.VectorSubcoreMesh(**_MESH),
    scratch_types=[
        pltpu.VMEM((80, ROW), jnp.int32),
        pltpu.VMEM((128,), jnp.float32),
        pltpu.VMEM((640,), jnp.float32),
        pltpu.VMEM_SHARED((NP,), jnp.float32),
    ],
)(_deg_body)


def _agg_body(g_hbm, src2d_hbm, dst2d_hbm, out_hbm,
              idxs_v, idxd_v, rows_v, agg_sh, sem):
    c = lax.axis_index("c")
    s = lax.axis_index("s")

    zero16 = jnp.zeros((16,), jnp.float32)

    def zrow(i, _):
        for k in range(8):
            rows_v[i, pl.ds(k * 16, 16)] = zero16
        return 0

    lax.fori_loop(0, 128, zrow, 0)
    for m in range(5):
        pltpu.sync_copy(rows_v, agg_sh.at[pl.ds(s * 640 + m * 128, 128)])
    plsc.subcore_barrier()

    rbase = (c * NS + s) * 80
    pltpu.sync_copy(src2d_hbm.at[pl.ds(rbase, 80)], idxs_v)
    pltpu.sync_copy(dst2d_hbm.at[pl.ds(rbase, 80)], idxd_v)

    def step(j, _):
        pltpu.async_copy(g_hbm.at[idxs_v.at[j]],
                         rows_v.at[pl.ds(0, ROW)], sem).wait()
        pltpu.sync_copy(rows_v.at[pl.ds(0, ROW)],
                        agg_sh.at[idxd_v.at[j]], add=True)
        return 0

    lax.fori_loop(0, 80, step, 0)

    plsc.subcore_barrier()
    pltpu.sync_copy(agg_sh.at[pl.ds(s * 640, 640)],
                    out_hbm.at[c, pl.ds(s * 640, 640)])


_agg_call = functools.partial(
    pl.kernel,
    out_type=jax.ShapeDtypeStruct((NC, NP, D), jnp.float32),
    mesh=plsc.VectorSubcoreMesh(**_MESH),
    scratch_types=[
        pltpu.VMEM((80, ROW), jnp.int32),
        pltpu.VMEM((80, ROW), jnp.int32),
        pltpu.VMEM((128, D), jnp.float32),
        pltpu.VMEM_SHARED((NP, D), jnp.float32),
        pltpu.SemaphoreType.DMA,
    ],
)(_agg_body)


def _edge_body(s1_hbm, s2_hbm, src_hbm, dst_hbm, out_hbm,
               s1_v, s2_v, srcb_v, dstb_v, outb_v):
    c = lax.axis_index("c")
    s = lax.axis_index("s")
    w = c * NS + s
    base = w * (E // (NC * NS))

    pltpu.sync_copy(s1_hbm, s1_v)
    pltpu.sync_copy(s2_hbm, s2_v)
    pltpu.sync_copy(src_hbm.at[pl.ds(base, 10000)], srcb_v)
    pltpu.sync_copy(dst_hbm.at[pl.ds(base, 10000)], dstb_v)

    def step(k, _):
        i1 = srcb_v[pl.ds(k * 16, 16)]
        i2 = dstb_v[pl.ds(k * 16, 16)]
        a = plsc.load_gather(s1_v, [i1])
        bb = plsc.load_gather(s2_v, [i2])
        outb_v[pl.ds(k * 16, 16)] = a + bb
        return 0

    lax.fori_loop(0, 625, step, 0)
    pltpu.sync_copy(outb_v, out_hbm.at[pl.ds(base, 10000)])


_edge_call = functools.partial(
    pl.kernel,
    out_type=jax.ShapeDtypeStruct((E,), jnp.float32),
    mesh=plsc.VectorSubcoreMesh(**_MESH),
    compiler_params=pltpu.CompilerParams(needs_layout_passes=False),
    scratch_types=[
        pltpu.VMEM((N,), jnp.float32),
        pltpu.VMEM((N,), jnp.float32),
        pltpu.VMEM((10000,), jnp.int32),
        pltpu.VMEM((10000,), jnp.int32),
        pltpu.VMEM((10000,), jnp.float32),
    ],
)(_edge_body)


def _mm_body(x_ref, w_ref, degp_ref, g_ref):
    i = pl.program_id(0)
    deg = degp_ref[0, pl.ds(i * 1280, 1280)] + degp_ref[1, pl.ds(i * 1280, 1280)]
    isd = lax.rsqrt(deg + 1.0)
    h = jnp.dot(x_ref[...], w_ref[...], preferred_element_type=jnp.float32)
    g_ref[...] = h * isd[:, None]


def _mm_call(x, w, degp):
    return pl.pallas_call(
        _mm_body,
        grid=(NP // 1280,),
        in_specs=[
            pl.BlockSpec((1280, D), lambda i: (i, 0)),
            pl.BlockSpec((D, D), lambda i: (0, 0)),
            pl.BlockSpec((NC, NP), lambda i: (0, 0)),
        ],
        out_specs=pl.BlockSpec((1280, D), lambda i: (i, 0)),
        out_shape=jax.ShapeDtypeStruct((N, D), jnp.float32),
    )(x, w, degp)


def _head_body(agg_ref, degp_ref, b_ref, wep_ref, bev_ref, out_ref):
    i = pl.program_id(0)
    deg = degp_ref[0, pl.ds(i * 1280, 1280)] + degp_ref[1, pl.ds(i * 1280, 1280)]
    isd = lax.rsqrt(deg + 1.0)
    agg = agg_ref[0] + agg_ref[1]
    h_out = jnp.maximum(agg * isd[:, None] + b_ref[0], 0.0)
    out_ref[...] = (
        jnp.dot(h_out, wep_ref[...], preferred_element_type=jnp.float32)
        + bev_ref[0]
    )


def _head_call(agg, degp, b2, wep, bev):
    return pl.pallas_call(
        _head_body,
        grid=(NP // 1280,),
        in_specs=[
            pl.BlockSpec((NC, 1280, D), lambda i: (0, i, 0)),
            pl.BlockSpec((NC, NP), lambda i: (0, 0)),
            pl.BlockSpec((1, D), lambda i: (0, 0)),
            pl.BlockSpec((D, D), lambda i: (0, 0)),
            pl.BlockSpec((1, D), lambda i: (0, 0)),
        ],
        out_specs=pl.BlockSpec((1280, D), lambda i: (i, 0)),
        out_shape=jax.ShapeDtypeStruct((N, D), jnp.float32),
    )(agg, degp, b2, wep, bev)


def kernel(x, edge_index, W, b, We, be):
    src = edge_index[0].astype(jnp.int32)
    dst = edge_index[1].astype(jnp.int32)
    src2d = src.reshape(EROWS, ROW)
    dst2d = dst.reshape(EROWS, ROW)

    degp = _deg_call(dst2d)
    g = _mm_call(x, W, degp)
    agg = _agg_call(g, src2d, dst2d)

    wep = jnp.zeros((D, D), jnp.float32)
    wep = wep.at[:, 0].set(We[:D, 0]).at[:, 1].set(We[D:, 0])
    bev = jnp.zeros((1, D), jnp.float32).at[0, 0].set(be[0])
    sout = _head_call(agg, degp, b.reshape(1, D), wep, bev)
    s1 = sout[:, 0]
    s2 = sout[:, 1]

    eout = _edge_call(s1, s2, src, dst)
    return eout[:, None]


# speedup vs baseline: 41.9401x; 1.3512x over previous
import functools

import jax
import jax.numpy as jnp
from jax import lax
from jax.experimental import pallas as pl
from jax.experimental.pallas import tpu as pltpu
from jax.experimental.pallas import tpu_sc as plsc

N = 10000
E = 320000
D = 128
F = D // 2
NP = 10240
NC = 2
NS = 16
ROW = 125
EROWS = E // ROW

_MESH = dict(core_axis_name="c", subcore_axis_name="s")


EPT = E // (NC * NS)


def _deg_body(dst_hbm, ones_hbm, out_hbm, idx_v, ones_v, zrow_v, hist_sh):
    c = lax.axis_index("c")
    s = lax.axis_index("s")

    zero16 = jnp.zeros((16,), jnp.float32)
    for i in range(40):
        zrow_v[pl.ds(i * 16, 16)] = zero16

    pltpu.sync_copy(zrow_v, hist_sh.at[pl.ds(s * 640, 640)])
    pltpu.sync_copy(ones_hbm, ones_v)
    plsc.subcore_barrier()

    w = c * NS + s
    pltpu.sync_copy(dst_hbm.at[w], idx_v)

    pltpu.sync_copy(ones_v, hist_sh.at[idx_v], add=True)
    plsc.subcore_barrier()
    pltpu.sync_copy(hist_sh.at[pl.ds(s * 640, 640)],
                    out_hbm.at[c, pl.ds(s * 640, 640)])


_deg_call = functools.partial(
    pl.kernel,
    out_type=jax.ShapeDtypeStruct((NC, NP), jnp.float32),
    mesh=plsc.VectorSubcoreMesh(**_MESH),
    scratch_types=[
        pltpu.VMEM((EPT,), jnp.int32),
        pltpu.VMEM((EPT,), jnp.float32),
        pltpu.VMEM((640,), jnp.float32),
        pltpu.VMEM_SHARED((NP,), jnp.float32),
    ],
)(_deg_body)


def _agg_body(g_hbm, src3_hbm, dst3_hbm, out_hbm,
              idxs_v, idxd_v, rows_v, agg_sh, sem):
    c = lax.axis_index("c")
    s = lax.axis_index("s")

    zero16 = jnp.zeros((16,), jnp.float32)

    def zrow(i, _):
        for k in range(8):
            rows_v[i, pl.ds(k * 16, 16)] = zero16
        return 0

    lax.fori_loop(0, 128, zrow, 0)
    for m in range(5):
        pltpu.sync_copy(rows_v.at[pl.ds(0, 128)],
                        agg_sh.at[pl.ds(s * 640 + m * 128, 128)])
    plsc.subcore_barrier()

    w = c * NS + s

    def gath(j, b):
        return pltpu.make_async_copy(
            g_hbm.at[idxs_v.at[j]],
            rows_v.at[pl.ds(b * 128, ROW)],
            sem.at[b],
        )

    def scat(j, b):
        pltpu.sync_copy(rows_v.at[pl.ds(b * 128, ROW)],
                        agg_sh.at[idxd_v.at[j]], add=True)

    for h in range(2):
        pltpu.sync_copy(src3_hbm.at[w, pl.ds(h * 40, 40)], idxs_v)
        pltpu.sync_copy(dst3_hbm.at[w, pl.ds(h * 40, 40)], idxd_v)
        gath(0, 0).start()
        gath(1, 1).start()

        def step(gi, _):
            for b in range(2):
                j = gi * 2 + b
                gath(j, b).wait()
                scat(j, b)

                @pl.when(j < 38)
                def _():
                    gath(j + 2, b).start()
            return 0

        lax.fori_loop(0, 20, step, 0)

    plsc.subcore_barrier()
    pltpu.sync_copy(agg_sh.at[pl.ds(s * 640, 640)],
                    out_hbm.at[c, pl.ds(s * 640, 640)])


_agg_call = functools.partial(
    pl.kernel,
    out_type=jax.ShapeDtypeStruct((NC, NP, D), jnp.float32),
    mesh=plsc.VectorSubcoreMesh(**_MESH),
    scratch_types=[
        pltpu.VMEM((40, ROW), jnp.int32),
        pltpu.VMEM((40, ROW), jnp.int32),
        pltpu.VMEM((256, D), jnp.float32),
        pltpu.VMEM_SHARED((NP, D), jnp.float32),
        pltpu.SemaphoreType.DMA((2,)),
    ],
)(_agg_body)


def _edge_body(s1_hbm, s2_hbm, src_hbm, dst_hbm, out_hbm,
               s1_v, s2_v, srcb_v, dstb_v, outb_v):
    c = lax.axis_index("c")
    s = lax.axis_index("s")
    w = c * NS + s
    base = w * (E // (NC * NS))

    pltpu.sync_copy(s1_hbm, s1_v)
    pltpu.sync_copy(s2_hbm, s2_v)
    pltpu.sync_copy(src_hbm.at[pl.ds(base, 10000)], srcb_v)
    pltpu.sync_copy(dst_hbm.at[pl.ds(base, 10000)], dstb_v)

    def step(k, _):
        i1 = srcb_v[pl.ds(k * 16, 16)]
        i2 = dstb_v[pl.ds(k * 16, 16)]
        a = plsc.load_gather(s1_v, [i1])
        bb = plsc.load_gather(s2_v, [i2])
        outb_v[pl.ds(k * 16, 16)] = a + bb
        return 0

    lax.fori_loop(0, 625, step, 0)
    pltpu.sync_copy(outb_v, out_hbm.at[pl.ds(base, 10000)])


_edge_call = functools.partial(
    pl.kernel,
    out_type=jax.ShapeDtypeStruct((E,), jnp.float32),
    mesh=plsc.VectorSubcoreMesh(**_MESH),
    compiler_params=pltpu.CompilerParams(needs_layout_passes=False),
    scratch_types=[
        pltpu.VMEM((N,), jnp.float32),
        pltpu.VMEM((N,), jnp.float32),
        pltpu.VMEM((10000,), jnp.int32),
        pltpu.VMEM((10000,), jnp.int32),
        pltpu.VMEM((10000,), jnp.float32),
    ],
)(_edge_body)


def _mm_body(x_ref, w_ref, degp_ref, g_ref):
    i = pl.program_id(0)
    deg = degp_ref[0, pl.ds(i * 1280, 1280)] + degp_ref[1, pl.ds(i * 1280, 1280)]
    isd = lax.rsqrt(deg + 1.0)
    h = jnp.dot(x_ref[...], w_ref[...], preferred_element_type=jnp.float32)
    g_ref[...] = h * isd[:, None]


def _mm_call(x, w, degp):
    return pl.pallas_call(
        _mm_body,
        grid=(NP // 1280,),
        in_specs=[
            pl.BlockSpec((1280, D), lambda i: (i, 0)),
            pl.BlockSpec((D, D), lambda i: (0, 0)),
            pl.BlockSpec((NC, NP), lambda i: (0, 0)),
        ],
        out_specs=pl.BlockSpec((1280, D), lambda i: (i, 0)),
        out_shape=jax.ShapeDtypeStruct((N, D), jnp.float32),
    )(x, w, degp)


def _head_body(agg_ref, degp_ref, b_ref, wep_ref, bev_ref, out_ref):
    i = pl.program_id(0)
    deg = degp_ref[0, pl.ds(i * 1280, 1280)] + degp_ref[1, pl.ds(i * 1280, 1280)]
    isd = lax.rsqrt(deg + 1.0)
    agg = agg_ref[0] + agg_ref[1]
    h_out = jnp.maximum(agg * isd[:, None] + b_ref[0], 0.0)
    out_ref[...] = (
        jnp.dot(h_out, wep_ref[...], preferred_element_type=jnp.float32)
        + bev_ref[0]
    )


def _head_call(agg, degp, b2, wep, bev):
    return pl.pallas_call(
        _head_body,
        grid=(NP // 1280,),
        in_specs=[
            pl.BlockSpec((NC, 1280, D), lambda i: (0, i, 0)),
            pl.BlockSpec((NC, NP), lambda i: (0, 0)),
            pl.BlockSpec((1, D), lambda i: (0, 0)),
            pl.BlockSpec((D, D), lambda i: (0, 0)),
            pl.BlockSpec((1, D), lambda i: (0, 0)),
        ],
        out_specs=pl.BlockSpec((1280, D), lambda i: (i, 0)),
        out_shape=jax.ShapeDtypeStruct((N, D), jnp.float32),
    )(agg, degp, b2, wep, bev)


def kernel(x, edge_index, W, b, We, be):
    src = edge_index[0].astype(jnp.int32)
    dst = edge_index[1].astype(jnp.int32)

    srcw = src.reshape(NC * NS, EPT)
    dstw = dst.reshape(NC * NS, EPT)
    src3 = src.reshape(NC * NS, EPT // ROW, ROW)
    dst3 = dst.reshape(NC * NS, EPT // ROW, ROW)
    ones1d = jnp.ones((EPT,), jnp.float32)
    degp = _deg_call(dstw, ones1d)
    g = _mm_call(x, W, degp)
    agg = _agg_call(g, src3, dst3)

    wep = jnp.zeros((D, D), jnp.float32)
    wep = wep.at[:, 0].set(We[:D, 0]).at[:, 1].set(We[D:, 0])
    bev = jnp.zeros((1, D), jnp.float32).at[0, 0].set(be[0])
    sout = _head_call(agg, degp, b.reshape(1, D), wep, bev)
    s1 = sout[:, 0]
    s2 = sout[:, 1]

    eout = _edge_call(s1, s2, src, dst)
    return eout[:, None]
